# baseline (speedup 1.0000x reference)
import functools

import jax
import jax.numpy as jnp
from jax import lax
from jax.experimental import pallas as pl
from jax.experimental.pallas import tpu as pltpu
from jax.experimental.pallas import tpu_sc as plsc

VOCAB_NO_POS = 999001
EMB = 64
SCALE = 8.0
LANES = 16

_info = plsc.get_sparse_core_info()
NC = _info.num_cores
NS = _info.num_subcores
NW = NC * NS


@jax.jit
def _embed(flat_tokens, table):
    B = flat_tokens.shape[0]
    b_per_w = B // NW
    C = 128
    n_chunks = b_per_w // C

    mesh = plsc.VectorSubcoreMesh(core_axis_name="c", subcore_axis_name="s")

    @functools.partial(
        pl.kernel,
        mesh=mesh,
        compiler_params=pltpu.CompilerParams(use_tc_tiling_on_sc=False),
        out_type=jax.ShapeDtypeStruct((B, EMB), jnp.float32),
        scratch_types=[
            pltpu.VMEM((b_per_w,), jnp.int32),
            pltpu.VMEM((C, EMB), jnp.float32),
            pltpu.SemaphoreType.DMA,
        ],
    )
    def body(tokens_hbm, table_hbm, out_hbm, idx_v, rows_v, sem):
        wid = lax.axis_index("s") * NC + lax.axis_index("c")
        base = wid * b_per_w
        pltpu.sync_copy(tokens_hbm.at[pl.ds(base, b_per_w)], idx_v)

        def clamp_body(i, carry):
            sl = pl.ds(i * LANES, LANES)
            idx_v[sl] = jnp.minimum(idx_v[sl], VOCAB_NO_POS - 1)
            return carry

        lax.fori_loop(0, b_per_w // LANES, clamp_body, 0)

        def chunk_body(g, carry):
            off = pl.multiple_of(g * C, C)
            pltpu.async_copy(
                table_hbm.at[idx_v.at[pl.ds(off, C)]], rows_v, sem
            ).wait()

            def scale_body(r, c2):
                for j in range(EMB // LANES):
                    sl = pl.ds(j * LANES, LANES)
                    rows_v[r, sl] = rows_v[r, sl] * SCALE
                return c2

            lax.fori_loop(0, C, scale_body, 0)
            pltpu.sync_copy(rows_v, out_hbm.at[pl.ds(base + off, C)])
            return carry

        lax.fori_loop(0, n_chunks, chunk_body, 0)

    return body(flat_tokens, table)


def kernel(tokens, table):
    S, T = tokens.shape
    flat = tokens.reshape(S * T).astype(jnp.int32)
    out = _embed(flat, table)
    return out.reshape(S, T, EMB)


# speedup vs baseline: 1.1273x; 1.1273x over previous
import functools

import jax
import jax.numpy as jnp
from jax import lax
from jax.experimental import pallas as pl
from jax.experimental.pallas import tpu as pltpu
from jax.experimental.pallas import tpu_sc as plsc

VOCAB_NO_POS = 999001
EMB = 64
SCALE = 8.0
LANES = 16

_info = plsc.get_sparse_core_info()
NC = _info.num_cores
NS = _info.num_subcores
NW = NC * NS


@jax.jit
def _embed(flat_tokens, table):
    B = flat_tokens.shape[0]
    b_per_w = B // NW
    C = 128
    n_chunks = b_per_w // C
    n_pairs = n_chunks // 2

    mesh = plsc.VectorSubcoreMesh(core_axis_name="c", subcore_axis_name="s")

    @functools.partial(
        pl.kernel,
        mesh=mesh,
        compiler_params=pltpu.CompilerParams(use_tc_tiling_on_sc=False),
        out_type=jax.ShapeDtypeStruct((B, EMB), jnp.float32),
        scratch_types=[
            pltpu.VMEM((b_per_w,), jnp.int32),
            pltpu.VMEM((C, EMB), jnp.float32),
            pltpu.VMEM((C, EMB), jnp.float32),
            pltpu.SemaphoreType.DMA,
            pltpu.SemaphoreType.DMA,
        ],
    )
    def body(tokens_hbm, table_hbm, out_hbm, idx_v, rows_a, rows_b, sem_a, sem_b):
        wid = lax.axis_index("s") * NC + lax.axis_index("c")
        base = wid * b_per_w
        pltpu.sync_copy(tokens_hbm.at[pl.ds(base, b_per_w)], idx_v)

        def _clamp(i, carry):
            sl = pl.ds(i * LANES, LANES)
            idx_v[sl] = jnp.minimum(idx_v[sl], VOCAB_NO_POS - 1)
            return carry

        lax.fori_loop(0, b_per_w // LANES, _clamp, 0)

        def start_gather(g, buf, sem):
            off = pl.multiple_of(g * C, C)
            pltpu.async_copy(table_hbm.at[idx_v.at[pl.ds(off, C)]], buf, sem)

        def wait_gather(buf, sem):
            pltpu.make_async_copy(table_hbm.at[pl.ds(0, C)], buf, sem).wait()

        def scale(buf):
            def _scale(r, carry):
                for j in range(EMB // LANES):
                    sl = pl.ds(j * LANES, LANES)
                    buf[r, sl] = buf[r, sl] * SCALE
                return carry

            lax.fori_loop(0, C, _scale, 0)

        def write_out(g, buf):
            off = pl.multiple_of(g * C, C)
            pltpu.sync_copy(buf, out_hbm.at[pl.ds(base + off, C)])

        start_gather(0, rows_a, sem_a)

        def pair_body(p, carry):
            g0 = p * 2

            wait_gather(rows_a, sem_a)
            start_gather(g0 + 1, rows_b, sem_b)
            scale(rows_a)
            write_out(g0, rows_a)

            wait_gather(rows_b, sem_b)

            @pl.when(p < n_pairs - 1)
            def _():
                start_gather(g0 + 2, rows_a, sem_a)

            scale(rows_b)
            write_out(g0 + 1, rows_b)
            return carry

        lax.fori_loop(0, n_pairs, pair_body, 0)

    return body(flat_tokens, table)


def kernel(tokens, table):
    S, T = tokens.shape
    flat = tokens.reshape(S * T).astype(jnp.int32)
    out = _embed(flat, table)
    return out.reshape(S, T, EMB)
